# baseline (speedup 1.0000x reference)
import functools

import jax
import jax.numpy as jnp
from jax import lax
from jax.experimental import pallas as pl
from jax.experimental.pallas import tpu as pltpu
from jax.experimental.pallas import tpu_sc as plsc

MAX_LEN = 8192
HIDDEN_DIM = 1024
NUM_CORES = 2
NUM_SUBCORES = 16
NUM_WORKERS = NUM_CORES * NUM_SUBCORES
ROWS_PER_WORKER = MAX_LEN // NUM_WORKERS


@functools.partial(
    pl.kernel,
    mesh=plsc.VectorSubcoreMesh(core_axis_name="c", subcore_axis_name="s"),
    out_type=jax.ShapeDtypeStruct((MAX_LEN, HIDDEN_DIM), jnp.float32),
    scratch_types=[
        pltpu.SemaphoreType.DMA,
    ],
)
def _pos_emb_copy(table_hbm, out_hbm, sem):
    wid = lax.axis_index("s") * NUM_CORES + lax.axis_index("c")
    base = wid * ROWS_PER_WORKER
    pltpu.async_copy(
        table_hbm.at[pl.ds(base, ROWS_PER_WORKER)],
        out_hbm.at[pl.ds(base, ROWS_PER_WORKER)],
        sem,
    ).wait()


def kernel(x, emb_table):
    seq_len = x.shape[1]
    out = _pos_emb_copy(emb_table)
    return out[None, :seq_len]


# speedup vs baseline: 24.1606x; 24.1606x over previous
import functools

import jax
import jax.numpy as jnp
from jax import lax
from jax.experimental import pallas as pl
from jax.experimental.pallas import tpu as pltpu
from jax.experimental.pallas import tpu_sc as plsc

MAX_LEN = 8192
HIDDEN_DIM = 1024
NUM_CORES = 2
NUM_SUBCORES = 16
NUM_WORKERS = NUM_CORES * NUM_SUBCORES
ROWS_PER_WORKER = MAX_LEN // NUM_WORKERS
CHUNK_ROWS = 32
NUM_CHUNKS = ROWS_PER_WORKER // CHUNK_ROWS
NUM_BUFS = 3


@functools.partial(
    pl.kernel,
    mesh=plsc.VectorSubcoreMesh(core_axis_name="c", subcore_axis_name="s"),
    out_type=jax.ShapeDtypeStruct((MAX_LEN, HIDDEN_DIM), jnp.float32),
    scratch_types=(
        [pltpu.VMEM((CHUNK_ROWS, HIDDEN_DIM), jnp.float32)] * NUM_BUFS
        + [pltpu.SemaphoreType.DMA] * (2 * NUM_BUFS)
    ),
)
def _pos_emb_copy(table_hbm, out_hbm, *scratch):
    bufs = scratch[:NUM_BUFS]
    in_sems = scratch[NUM_BUFS:2 * NUM_BUFS]
    out_sems = scratch[2 * NUM_BUFS:]
    wid = lax.axis_index("s") * NUM_CORES + lax.axis_index("c")
    base = wid * ROWS_PER_WORKER

    in_d = [None] * NUM_CHUNKS
    out_d = [None] * NUM_CHUNKS
    for i in range(NUM_CHUNKS):
        b = i % NUM_BUFS
        if i >= NUM_BUFS:
            out_d[i - NUM_BUFS].wait()
        r0 = base + i * CHUNK_ROWS
        in_d[i] = pltpu.async_copy(
            table_hbm.at[pl.ds(r0, CHUNK_ROWS)], bufs[b], in_sems[b])
        if i >= 1:
            j = i - 1
            in_d[j].wait()
            out_d[j] = pltpu.async_copy(
                bufs[j % NUM_BUFS],
                out_hbm.at[pl.ds(base + j * CHUNK_ROWS, CHUNK_ROWS)],
                out_sems[j % NUM_BUFS])
    last = NUM_CHUNKS - 1
    in_d[last].wait()
    out_d[last] = pltpu.async_copy(
        bufs[last % NUM_BUFS],
        out_hbm.at[pl.ds(base + last * CHUNK_ROWS, CHUNK_ROWS)],
        out_sems[last % NUM_BUFS])
    for j in range(NUM_CHUNKS - NUM_BUFS, NUM_CHUNKS):
        out_d[j].wait()


def kernel(x, emb_table):
    seq_len = x.shape[1]
    out = _pos_emb_copy(emb_table)
    return out[None, :seq_len]
